# baseline (speedup 1.0000x reference)
import functools

import jax
import jax.numpy as jnp
from jax import lax
from jax.experimental import pallas as pl
from jax.experimental.pallas import tpu as pltpu
from jax.experimental.pallas import tpu_sc as plsc

N = 10000
E = 160000
D = 256
H = 128
NC = 2
NS = 16
EP = 163840
PAD_DST = 10200
NP = 10240
DEGW = 16

_mesh = plsc.VectorSubcoreMesh(core_axis_name="c", subcore_axis_name="s")


@functools.partial(
    pl.kernel,
    out_type=(
        jax.ShapeDtypeStruct((NP, DEGW), jnp.float32),
        jax.ShapeDtypeStruct((NP, DEGW), jnp.float32),
    ),
    mesh=_mesh,
    scratch_types=(
        pltpu.VMEM((40, 128), jnp.int32),
        pltpu.VMEM((128, DEGW), jnp.float32),
        pltpu.VMEM_SHARED((NP, DEGW), jnp.float32),
    ),
)
def _deg_kernel(dst_hbm, deg0_hbm, deg1_hbm, dst_v, wbuf, deg_sh):
    cid = lax.axis_index("c")
    sid = lax.axis_index("s")
    wid = cid * NS + sid

    def fill(val):
        def body(i, _):
            wbuf[i, pl.ds(0, 16)] = jnp.full((16,), val, jnp.float32)
            return 0
        lax.fori_loop(0, 128, body, 0)

    fill(0.0)
    def zero_body(k, _):
        pltpu.sync_copy(wbuf, deg_sh.at[pl.ds(sid * 640 + k * 128, 128)])
        return 0
    lax.fori_loop(0, 5, zero_body, 0)
    plsc.subcore_barrier()

    pltpu.sync_copy(dst_hbm.at[wid], dst_v)
    fill(1.0)
    def add_body(j, _):
        pltpu.sync_copy(wbuf, deg_sh.at[dst_v.at[j]], add=True)
        return 0
    lax.fori_loop(0, 40, add_body, 0)
    plsc.subcore_barrier()

    def out_body(k, _):
        rows = pl.ds(sid * 640 + k * 128, 128)
        pltpu.sync_copy(deg_sh.at[rows], wbuf)
        @pl.when(cid == 0)
        def _():
            pltpu.sync_copy(wbuf, deg0_hbm.at[rows])
        @pl.when(cid == 1)
        def _():
            pltpu.sync_copy(wbuf, deg1_hbm.at[rows])
        return 0
    lax.fori_loop(0, 5, out_body, 0)


CH = 512
NCH = 10240 // CH


@functools.partial(
    pl.kernel,
    out_type=(
        jax.ShapeDtypeStruct((NP, H), jnp.float32),
        jax.ShapeDtypeStruct((NP, H), jnp.float32),
    ),
    mesh=_mesh,
    scratch_types=(
        pltpu.VMEM((80, 128), jnp.int32),
        pltpu.VMEM((80, 128), jnp.int32),
        pltpu.VMEM((128, H), jnp.float32),
        pltpu.VMEM_SHARED((NP, H), jnp.float32),
    ),
)
def _segsum_kernel(src_hbm, dst_hbm, hp0_hbm, hp1_hbm, ss0_hbm, ss1_hbm,
                   src_v, dst_v, gbuf, acc_sh):
    cid = lax.axis_index("c")
    sid = lax.axis_index("s")

    def zfill(i, _):
        def seg(k, _):
            gbuf[i, pl.ds(k * 16, 16)] = jnp.zeros((16,), jnp.float32)
            return 0
        lax.fori_loop(0, H // 16, seg, 0)
        return 0
    lax.fori_loop(0, 128, zfill, 0)
    def zero_body(k, _):
        pltpu.sync_copy(gbuf, acc_sh.at[pl.ds(sid * 640 + k * 128, 128)])
        return 0
    lax.fori_loop(0, 5, zero_body, 0)
    plsc.subcore_barrier()

    pltpu.sync_copy(src_hbm.at[pl.ds(sid * 80, 80)], src_v)
    pltpu.sync_copy(dst_hbm.at[pl.ds(sid * 80, 80)], dst_v)

    def run(hp_hbm):
        def body(j, _):
            pltpu.sync_copy(hp_hbm.at[src_v.at[j]], gbuf)
            pltpu.sync_copy(gbuf, acc_sh.at[dst_v.at[j]], add=True)
            return 0
        lax.fori_loop(0, 80, body, 0)

    @pl.when(cid == 0)
    def _():
        run(hp0_hbm)
    @pl.when(cid == 1)
    def _():
        run(hp1_hbm)
    plsc.subcore_barrier()

    def out_body(k, _):
        rows = pl.ds(sid * 640 + k * 128, 128)
        pltpu.sync_copy(acc_sh.at[rows], gbuf)
        @pl.when(cid == 0)
        def _():
            pltpu.sync_copy(gbuf, ss0_hbm.at[rows])
        @pl.when(cid == 1)
        def _():
            pltpu.sync_copy(gbuf, ss1_hbm.at[rows])
        return 0
    lax.fori_loop(0, 5, out_body, 0)


BS = 1000


def _tc1_body(x_ref, w0_ref, w1_ref, d0_ref, d1_ref, hp0_ref, hp1_ref):
    deg = d0_ref[:, :1] + d1_ref[:, :1] + 1.0
    ds = lax.rsqrt(deg)
    xb = x_ref[...]
    dot = functools.partial(jnp.dot, preferred_element_type=jnp.float32,
                            precision=lax.Precision.HIGHEST)
    hp0_ref[...] = dot(xb, w0_ref[...]) * ds
    hp1_ref[...] = dot(xb, w1_ref[...]) * ds


def _tc2a_body(ss0, ss1, hp0, hp1, d0, d1, b0, b1, t_ref, s0, q0, s1, q1):
    deg = d0[:, :1] + d1[:, :1] + 1.0
    ds = lax.rsqrt(deg)
    a0 = ds * (ss0[...] + hp0[...]) + b0[...]
    a1 = ds * (ss1[...] + hp1[...]) + b1[...]
    t0 = jnp.where(a0 >= 0, a0, 0.01 * a0)
    t1 = jnp.where(a1 >= 0, a1, 0.01 * a1)
    t_ref[:, :H] = t0
    t_ref[:, H:] = t1

    @pl.when(pl.program_id(0) == 0)
    def _():
        s0[...] = jnp.zeros_like(s0)
        q0[...] = jnp.zeros_like(q0)
        s1[...] = jnp.zeros_like(s1)
        q1[...] = jnp.zeros_like(q1)

    s0[...] += jnp.sum(t0, axis=0, keepdims=True)
    q0[...] += jnp.sum(t0 * t0, axis=0, keepdims=True)
    s1[...] += jnp.sum(t1, axis=0, keepdims=True)
    q1[...] += jnp.sum(t1 * t1, axis=0, keepdims=True)


def _tc2b_body(t, s0, q0, s1, q1, g0, g1, be0, be1, out_ref):
    n = jnp.float32(N)
    m0 = s0[...] / n
    m1 = s1[...] / n
    v0 = q0[...] / n - m0 * m0
    v1 = q1[...] / n - m1 * m1
    sc0 = g0[...] * lax.rsqrt(v0 + 1e-5)
    sc1 = g1[...] * lax.rsqrt(v1 + 1e-5)
    out_ref[:, :H] = t[:, :H] * sc0 + (be0[...] - m0 * sc0)
    out_ref[:, H:] = t[:, H:] * sc1 + (be1[...] - m1 * sc1)


def kernel(x, edge_index, W, b, gamma, beta):
    src = edge_index[0].astype(jnp.int32)
    dst = edge_index[1].astype(jnp.int32)
    npad = EP - E
    src_p = jnp.concatenate([src, jnp.zeros((npad,), jnp.int32)])
    dst_p = jnp.concatenate([dst, jnp.full((npad,), PAD_DST, jnp.int32)])
    dst3 = dst_p.reshape(NC * NS, 40, 128)
    src2 = src_p.reshape(NS * 80, 128)
    dst2 = dst_p.reshape(NS * 80, 128)

    deg0, deg1 = _deg_kernel(dst3)

    grid = N // BS
    row_block = lambda w: pl.BlockSpec((BS, w), lambda i: (i, 0))
    full = lambda shp: pl.BlockSpec(shp, lambda i: tuple(0 for _ in shp))

    hp0, hp1 = pl.pallas_call(
        _tc1_body,
        grid=(grid,),
        in_specs=[row_block(D), full((D, H)), full((D, H)),
                  row_block(DEGW), row_block(DEGW)],
        out_specs=[row_block(H), row_block(H)],
        out_shape=[jax.ShapeDtypeStruct((N, H), jnp.float32)] * 2,
    )(x, W[:, :H], W[:, H:], deg0, deg1)

    ss0, ss1 = _segsum_kernel(src2, dst2, hp0, hp1)

    stat = pl.BlockSpec((1, H), lambda i: (0, 0))
    t, s0, q0, s1, q1 = pl.pallas_call(
        _tc2a_body,
        grid=(grid,),
        in_specs=[row_block(H)] * 4 + [row_block(DEGW)] * 2 + [full((1, H))] * 2,
        out_specs=[row_block(D), stat, stat, stat, stat],
        out_shape=[jax.ShapeDtypeStruct((N, D), jnp.float32)]
        + [jax.ShapeDtypeStruct((1, H), jnp.float32)] * 4,
    )(ss0, ss1, hp0, hp1, deg0, deg1,
      b[:H].reshape(1, H), b[H:].reshape(1, H))

    out = pl.pallas_call(
        _tc2b_body,
        grid=(grid,),
        in_specs=[row_block(D)] + [full((1, H))] * 8,
        out_specs=row_block(D),
        out_shape=jax.ShapeDtypeStruct((N, D), jnp.float32),
    )(t, s0, q0, s1, q1,
      gamma[:H].reshape(1, H), gamma[H:].reshape(1, H),
      beta[:H].reshape(1, H), beta[H:].reshape(1, H))
    return out


# speedup vs baseline: 1.0015x; 1.0015x over previous
import functools

import jax
import jax.numpy as jnp
from jax import lax
from jax.experimental import pallas as pl
from jax.experimental.pallas import tpu as pltpu
from jax.experimental.pallas import tpu_sc as plsc

N = 10000
E = 160000
D = 256
H = 128
NC = 2
NS = 16
EP = 163840
PAD_DST = 10200
NP = 10240
DEGW = 16

_mesh = plsc.VectorSubcoreMesh(core_axis_name="c", subcore_axis_name="s")


@functools.partial(
    pl.kernel,
    out_type=(
        jax.ShapeDtypeStruct((NP, DEGW), jnp.float32),
        jax.ShapeDtypeStruct((NP, DEGW), jnp.float32),
    ),
    mesh=_mesh,
    scratch_types=(
        pltpu.VMEM((40, 128), jnp.int32),
        pltpu.VMEM((128, DEGW), jnp.float32),
        pltpu.VMEM_SHARED((NP, DEGW), jnp.float32),
    ),
)
def _deg_kernel(dst_hbm, deg0_hbm, deg1_hbm, dst_v, wbuf, deg_sh):
    cid = lax.axis_index("c")
    sid = lax.axis_index("s")
    wid = cid * NS + sid

    def fill(val):
        def body(i, _):
            wbuf[i, pl.ds(0, 16)] = jnp.full((16,), val, jnp.float32)
            return 0
        lax.fori_loop(0, 128, body, 0)

    fill(0.0)
    def zero_body(k, _):
        pltpu.sync_copy(wbuf, deg_sh.at[pl.ds(sid * 640 + k * 128, 128)])
        return 0
    lax.fori_loop(0, 5, zero_body, 0)
    plsc.subcore_barrier()

    pltpu.sync_copy(dst_hbm.at[wid], dst_v)
    fill(1.0)
    def add_body(j, _):
        pltpu.sync_copy(wbuf, deg_sh.at[dst_v.at[j]], add=True)
        return 0
    lax.fori_loop(0, 40, add_body, 0)
    plsc.subcore_barrier()
    plsc.subcore_barrier()

    def out_body(k, _):
        rows = pl.ds(sid * 640 + k * 128, 128)
        pltpu.sync_copy(deg_sh.at[rows], wbuf)
        @pl.when(cid == 0)
        def _():
            pltpu.sync_copy(wbuf, deg0_hbm.at[rows])
        @pl.when(cid == 1)
        def _():
            pltpu.sync_copy(wbuf, deg1_hbm.at[rows])
        return 0
    lax.fori_loop(0, 5, out_body, 0)


CH = 512
NCH = 10240 // CH


@functools.partial(
    pl.kernel,
    out_type=(
        jax.ShapeDtypeStruct((NP, H), jnp.float32),
        jax.ShapeDtypeStruct((NP, H), jnp.float32),
    ),
    mesh=_mesh,
    scratch_types=(
        pltpu.VMEM((80, 128), jnp.int32),
        pltpu.VMEM((80, 128), jnp.int32),
        pltpu.VMEM((128, H), jnp.float32),
        pltpu.VMEM_SHARED((NP, H), jnp.float32),
    ),
)
def _segsum_kernel(src_hbm, dst_hbm, hp0_hbm, hp1_hbm, ss0_hbm, ss1_hbm,
                   src_v, dst_v, gbuf, acc_sh):
    cid = lax.axis_index("c")
    sid = lax.axis_index("s")

    def zfill(i, _):
        def seg(k, _):
            gbuf[i, pl.ds(k * 16, 16)] = jnp.zeros((16,), jnp.float32)
            return 0
        lax.fori_loop(0, H // 16, seg, 0)
        return 0
    lax.fori_loop(0, 128, zfill, 0)
    def zero_body(k, _):
        pltpu.sync_copy(gbuf, acc_sh.at[pl.ds(sid * 640 + k * 128, 128)])
        return 0
    lax.fori_loop(0, 5, zero_body, 0)
    plsc.subcore_barrier()

    pltpu.sync_copy(src_hbm.at[pl.ds(sid * 80, 80)], src_v)
    pltpu.sync_copy(dst_hbm.at[pl.ds(sid * 80, 80)], dst_v)

    def run(hp_hbm):
        def body(j, _):
            pltpu.sync_copy(hp_hbm.at[src_v.at[j]], gbuf)
            pltpu.sync_copy(gbuf, acc_sh.at[dst_v.at[j]], add=True)
            return 0
        lax.fori_loop(0, 80, body, 0)

    @pl.when(cid == 0)
    def _():
        run(hp0_hbm)
    @pl.when(cid == 1)
    def _():
        run(hp1_hbm)
    plsc.subcore_barrier()
    plsc.subcore_barrier()

    def out_body(k, _):
        rows = pl.ds(sid * 640 + k * 128, 128)
        pltpu.sync_copy(acc_sh.at[rows], gbuf)
        @pl.when(cid == 0)
        def _():
            pltpu.sync_copy(gbuf, ss0_hbm.at[rows])
        @pl.when(cid == 1)
        def _():
            pltpu.sync_copy(gbuf, ss1_hbm.at[rows])
        return 0
    lax.fori_loop(0, 5, out_body, 0)


BS = 1000


def _tc1_body(x_ref, w0_ref, w1_ref, d0_ref, d1_ref, hp0_ref, hp1_ref):
    deg = d0_ref[:, :1] + d1_ref[:, :1] + 1.0
    ds = lax.rsqrt(deg)
    xb = x_ref[...]
    dot = functools.partial(jnp.dot, preferred_element_type=jnp.float32,
                            precision=lax.Precision.HIGHEST)
    hp0_ref[...] = dot(xb, w0_ref[...]) * ds
    hp1_ref[...] = dot(xb, w1_ref[...]) * ds


def _tc2a_body(ss0, ss1, hp0, hp1, d0, d1, b0, b1, t_ref, s0, q0, s1, q1):
    deg = d0[:, :1] + d1[:, :1] + 1.0
    ds = lax.rsqrt(deg)
    a0 = ds * (ss0[...] + hp0[...]) + b0[...]
    a1 = ds * (ss1[...] + hp1[...]) + b1[...]
    t0 = jnp.where(a0 >= 0, a0, 0.01 * a0)
    t1 = jnp.where(a1 >= 0, a1, 0.01 * a1)
    t_ref[:, :H] = t0
    t_ref[:, H:] = t1

    @pl.when(pl.program_id(0) == 0)
    def _():
        s0[...] = jnp.zeros_like(s0)
        q0[...] = jnp.zeros_like(q0)
        s1[...] = jnp.zeros_like(s1)
        q1[...] = jnp.zeros_like(q1)

    s0[...] += jnp.sum(t0, axis=0, keepdims=True)
    q0[...] += jnp.sum(t0 * t0, axis=0, keepdims=True)
    s1[...] += jnp.sum(t1, axis=0, keepdims=True)
    q1[...] += jnp.sum(t1 * t1, axis=0, keepdims=True)


def _tc2b_body(t, s0, q0, s1, q1, g0, g1, be0, be1, out_ref):
    n = jnp.float32(N)
    m0 = s0[...] / n
    m1 = s1[...] / n
    v0 = q0[...] / n - m0 * m0
    v1 = q1[...] / n - m1 * m1
    sc0 = g0[...] * lax.rsqrt(v0 + 1e-5)
    sc1 = g1[...] * lax.rsqrt(v1 + 1e-5)
    out_ref[:, :H] = t[:, :H] * sc0 + (be0[...] - m0 * sc0)
    out_ref[:, H:] = t[:, H:] * sc1 + (be1[...] - m1 * sc1)


def kernel(x, edge_index, W, b, gamma, beta):
    src = edge_index[0].astype(jnp.int32)
    dst = edge_index[1].astype(jnp.int32)
    npad = EP - E
    src_p = jnp.concatenate([src, jnp.zeros((npad,), jnp.int32)])
    dst_p = jnp.concatenate([dst, jnp.full((npad,), PAD_DST, jnp.int32)])
    dst3 = dst_p.reshape(NC * NS, 40, 128)
    src2 = src_p.reshape(NS * 80, 128)
    dst2 = dst_p.reshape(NS * 80, 128)

    deg0, deg1 = _deg_kernel(dst3)

    grid = N // BS
    row_block = lambda w: pl.BlockSpec((BS, w), lambda i: (i, 0))
    full = lambda shp: pl.BlockSpec(shp, lambda i: tuple(0 for _ in shp))

    hp0, hp1 = pl.pallas_call(
        _tc1_body,
        grid=(grid,),
        in_specs=[row_block(D), full((D, H)), full((D, H)),
                  row_block(DEGW), row_block(DEGW)],
        out_specs=[row_block(H), row_block(H)],
        out_shape=[jax.ShapeDtypeStruct((N, H), jnp.float32)] * 2,
    )(x, W[:, :H], W[:, H:], deg0, deg1)

    ss0, ss1 = _segsum_kernel(src2, dst2, hp0, hp1)

    stat = pl.BlockSpec((1, H), lambda i: (0, 0))
    t, s0, q0, s1, q1 = pl.pallas_call(
        _tc2a_body,
        grid=(grid,),
        in_specs=[row_block(H)] * 4 + [row_block(DEGW)] * 2 + [full((1, H))] * 2,
        out_specs=[row_block(D), stat, stat, stat, stat],
        out_shape=[jax.ShapeDtypeStruct((N, D), jnp.float32)]
        + [jax.ShapeDtypeStruct((1, H), jnp.float32)] * 4,
    )(ss0, ss1, hp0, hp1, deg0, deg1,
      b[:H].reshape(1, H), b[H:].reshape(1, H))

    out = pl.pallas_call(
        _tc2b_body,
        grid=(grid,),
        in_specs=[row_block(D)] + [full((1, H))] * 8,
        out_specs=row_block(D),
        out_shape=jax.ShapeDtypeStruct((N, D), jnp.float32),
    )(t, s0, q0, s1, q1,
      gamma[:H].reshape(1, H), gamma[H:].reshape(1, H),
      beta[:H].reshape(1, H), beta[H:].reshape(1, H))
    return out
